# baseline (speedup 1.0000x reference)
import jax
import jax.numpy as jnp
from jax.experimental import pallas as pl

_NUM_CLASSES = 1000
_BATCH = 16384
_FEAT = 100
_OUT_H = _NUM_CLASSES + _FEAT
_BN = 1024


def _onehot_concat_kernel(xt_ref, o_ref):
    xb = xt_ref[...]
    sel = xb[0:1, :].astype(jnp.int32)
    rows = jax.lax.broadcasted_iota(jnp.int32, (_NUM_CLASSES, _BN), 0)
    oh = (rows == sel).astype(xb.dtype)
    o_ref[...] = jnp.concatenate([oh, xb], axis=0)


def kernel(x):
    xt = x.T
    grid = (_BATCH // _BN,)
    out_t = pl.pallas_call(
        _onehot_concat_kernel,
        grid=grid,
        in_specs=[pl.BlockSpec((_FEAT, _BN), lambda i: (0, i))],
        out_specs=pl.BlockSpec((_OUT_H, _BN), lambda i: (0, i)),
        out_shape=jax.ShapeDtypeStruct((_OUT_H, _BATCH), x.dtype),
    )(xt)
    return out_t.T


# speedup vs baseline: 1.0604x; 1.0604x over previous
import jax
import jax.numpy as jnp
from jax.experimental import pallas as pl

_NUM_CLASSES = 1000
_BN = 2048


def _onehot_concat_kernel(xt_ref, o_ref):
    xb = xt_ref[...]
    sel = xb[0:1, :].astype(jnp.int32)
    rows = jax.lax.broadcasted_iota(jnp.int32, (_NUM_CLASSES, _BN), 0)
    oh = (rows == sel).astype(xb.dtype)
    o_ref[...] = jnp.concatenate([oh, xb], axis=0)


def kernel(x):
    batch, feat = x.shape
    out_h = _NUM_CLASSES + feat
    xt = x.T
    out_t = pl.pallas_call(
        _onehot_concat_kernel,
        grid=(batch // _BN,),
        in_specs=[pl.BlockSpec((feat, _BN), lambda i: (0, i))],
        out_specs=pl.BlockSpec((out_h, _BN), lambda i: (0, i)),
        out_shape=jax.ShapeDtypeStruct((out_h, batch), x.dtype),
    )(xt)
    return out_t.T
